# Initial kernel scaffold; baseline (speedup 1.0000x reference)
import jax
import jax.numpy as jnp
from jax.experimental import pallas as pl


def kernel(hidden_states, gate_w, w_gate, w_up, w_down):
    raise NotImplementedError("write your pallas kernel here")


# speedup vs baseline: 1.5618x; 1.5618x over previous
import functools

import jax
import jax.numpy as jnp
from jax.experimental import pallas as pl
from jax.experimental.pallas import tpu as pltpu

E = 8
TOP_K = 2
D_MODEL = 1024
D_FF = 1024
T = 2048
E_PAD = 128
T_TILE = 512
NEG = -1e30


def _router_body(x_ref, gw_ref, ids_ref, combine_ref):
    logits = jnp.dot(x_ref[...], gw_ref[...],
                     preferred_element_type=jnp.float32)
    col = jax.lax.broadcasted_iota(jnp.int32, logits.shape, 1)
    logits = jnp.where(col < E, logits, NEG)
    m1 = jnp.max(logits, axis=1, keepdims=True)
    id1 = jnp.min(jnp.where(logits == m1, col, E_PAD), axis=1, keepdims=True)
    logits2 = jnp.where(col == id1, NEG, logits)
    m2 = jnp.max(logits2, axis=1, keepdims=True)
    id2 = jnp.min(jnp.where(logits2 == m2, col, E_PAD), axis=1, keepdims=True)
    w1 = 1.0 / (1.0 + jnp.exp(m2 - m1))
    w2 = 1.0 - w1
    ids_ref[:, 0:1] = id1
    ids_ref[:, 1:2] = id2
    ecol = jax.lax.broadcasted_iota(jnp.int32, (T, E), 1)
    combine_ref[...] = jnp.where(ecol == id1, w1, 0.0) + jnp.where(ecol == id2, w2, 0.0)


def _moe_body(x_ref, combine_ref, wg_ref, wu_ref, wd_ref, out_ref):
    e = pl.program_id(0)
    t = pl.program_id(1)
    rows = pl.ds(t * T_TILE, T_TILE)
    x = x_ref[rows, :]
    hg = jnp.dot(x, wg_ref[0], preferred_element_type=jnp.float32)
    hu = jnp.dot(x, wu_ref[0], preferred_element_type=jnp.float32)
    h = hg / (1.0 + jnp.exp(-hg)) * hu
    y = jnp.dot(h, wd_ref[0], preferred_element_type=jnp.float32)
    ecol = jax.lax.broadcasted_iota(jnp.int32, (T_TILE, E), 1)
    w = jnp.sum(jnp.where(ecol == e, combine_ref[rows, :], 0.0), axis=1,
                keepdims=True)
    contrib = y * w

    @pl.when(e == 0)
    def _():
        out_ref[rows, :] = contrib

    @pl.when(e != 0)
    def _():
        out_ref[rows, :] = out_ref[rows, :] + contrib


@jax.jit
def kernel(hidden_states, gate_w, w_gate, w_up, w_down):
    gw_pad = jnp.zeros((D_MODEL, E_PAD), jnp.float32).at[:, :E].set(gate_w)
    topk_ids, combine = pl.pallas_call(
        _router_body,
        out_shape=(
            jax.ShapeDtypeStruct((T, TOP_K), jnp.int32),
            jax.ShapeDtypeStruct((T, E), jnp.float32),
        ),
    )(hidden_states, gw_pad)

    out = pl.pallas_call(
        _moe_body,
        grid=(E, T // T_TILE),
        in_specs=[
            pl.BlockSpec((T, D_MODEL), lambda e, t: (0, 0)),
            pl.BlockSpec((T, E), lambda e, t: (0, 0)),
            pl.BlockSpec((1, D_MODEL, D_FF), lambda e, t: (e, 0, 0)),
            pl.BlockSpec((1, D_MODEL, D_FF), lambda e, t: (e, 0, 0)),
            pl.BlockSpec((1, D_FF, D_MODEL), lambda e, t: (e, 0, 0)),
        ],
        out_specs=pl.BlockSpec((T, D_MODEL), lambda e, t: (0, 0)),
        out_shape=jax.ShapeDtypeStruct((T, D_MODEL), jnp.float32),
    )(hidden_states, combine, w_gate, w_up, w_down)
    return (out, topk_ids)
